# baseline (speedup 1.0000x reference)
import jax
import jax.numpy as jnp
from jax import lax
from jax.experimental import pallas as pl
from jax.experimental.pallas import tpu as pltpu


def _make_body(f, h, w):
    hw = h * w

    def body(xe_ref, ye_ref, o_ref):
        k = lax.broadcasted_iota(jnp.int32, (w, hw), 1)
        r = lax.broadcasted_iota(jnp.int32, (w, hw), 0)
        sel_x = (k % w == r).astype(jnp.float32)
        sel_y = (k // w == r).astype(jnp.float32)
        dn = (((0,), (0,)), ((), ()))
        o_ref[0, :f] = lax.dot_general(
            xe_ref[...], sel_x, dn, preferred_element_type=jnp.float32)
        o_ref[0, f:] = lax.dot_general(
            ye_ref[...], sel_y, dn, preferred_element_type=jnp.float32)

    return body


def kernel(x, xenc, yenc):
    b = x.shape[0]
    h, w = x.shape[-2], x.shape[-1]
    f = xenc.shape[1]
    out = pl.pallas_call(
        _make_body(f, h, w),
        grid=(b,),
        in_specs=[
            pl.BlockSpec((w, f), lambda i: (0, 0)),
            pl.BlockSpec((h, f), lambda i: (0, 0)),
        ],
        out_specs=pl.BlockSpec((1, 2 * f, h * w), lambda i: (i, 0, 0)),
        out_shape=jax.ShapeDtypeStruct((b, 2 * f, h * w), jnp.float32),
    )(xenc[:w], yenc[:h])
    return out.reshape(b, 2 * f, h, w)


# speedup vs baseline: 1.0761x; 1.0761x over previous
import jax
import jax.numpy as jnp
from jax import lax
from jax.experimental import pallas as pl
from jax.experimental.pallas import tpu as pltpu

_REP = 4


def _make_body(b, f, h, w, rep):
    hw = h * w
    n_chunks = b // rep

    def body(xe_ref, ye_ref, o_ref, scratch_ref, sem):
        k = lax.broadcasted_iota(jnp.int32, (w, hw), 1)
        r = lax.broadcasted_iota(jnp.int32, (w, hw), 0)
        sel_x = (k % w == r).astype(jnp.float32)
        sel_y = (k // w == r).astype(jnp.float32)
        dn = (((0,), (0,)), ((), ()))
        xrow = lax.dot_general(
            xe_ref[...], sel_x, dn, preferred_element_type=jnp.float32)
        yrow = lax.dot_general(
            ye_ref[...], sel_y, dn, preferred_element_type=jnp.float32)
        for i in range(rep):
            scratch_ref[i, :f] = xrow
            scratch_ref[i, f:] = yrow
        for g in range(n_chunks):
            pltpu.make_async_copy(
                scratch_ref, o_ref.at[pl.ds(g * rep, rep)], sem.at[g]).start()
        for g in range(n_chunks):
            pltpu.make_async_copy(
                scratch_ref, o_ref.at[pl.ds(g * rep, rep)], sem.at[g]).wait()

    return body


def kernel(x, xenc, yenc):
    b = x.shape[0]
    h, w = x.shape[-2], x.shape[-1]
    f = xenc.shape[1]
    rep = _REP if b % _REP == 0 else 1
    out = pl.pallas_call(
        _make_body(b, f, h, w, rep),
        in_specs=[
            pl.BlockSpec(memory_space=pltpu.MemorySpace.VMEM),
            pl.BlockSpec(memory_space=pltpu.MemorySpace.VMEM),
        ],
        out_specs=pl.BlockSpec(memory_space=pltpu.MemorySpace.HBM),
        out_shape=jax.ShapeDtypeStruct((b, 2 * f, h * w), jnp.float32),
        scratch_shapes=[
            pltpu.VMEM((rep, 2 * f, h * w), jnp.float32),
            pltpu.SemaphoreType.DMA((b // rep,)),
        ],
    )(xenc[:w], yenc[:h])
    return out.reshape(b, 2 * f, h, w)
